# baseline (speedup 1.0000x reference)
import jax
import jax.numpy as jnp
from jax.experimental import pallas as pl
from jax.experimental.pallas import tpu as pltpu

TOKENS = 8192
HIDDEN = 2048
EXPERTS = 16
BT = 1024


def _body(x_ref, w_ref, logits_ref, idx_ref, wt_ref):
    @pl.when(pl.program_id(0) == 0)
    def _():
        wt_ref[...] = w_ref[...].T

    xb = x_ref[...]
    l = jnp.dot(xb, wt_ref[...], preferred_element_type=jnp.float32)
    logits_ref[...] = l
    m = jnp.max(l, axis=-1, keepdims=True)
    e_iota = jax.lax.broadcasted_iota(jnp.int32, (BT, EXPERTS), 1)
    idx_ref[...] = jnp.min(jnp.where(l == m, e_iota, EXPERTS), axis=-1)


def kernel(x, W):
    logits, idx = pl.pallas_call(
        _body,
        grid=(TOKENS // BT,),
        in_specs=[
            pl.BlockSpec((BT, HIDDEN), lambda i: (i, 0)),
            pl.BlockSpec((EXPERTS, HIDDEN), lambda i: (0, 0)),
        ],
        out_specs=[
            pl.BlockSpec((BT, EXPERTS), lambda i: (i, 0)),
            pl.BlockSpec((BT,), lambda i: (i,)),
        ],
        out_shape=[
            jax.ShapeDtypeStruct((TOKENS, EXPERTS), jnp.float32),
            jax.ShapeDtypeStruct((TOKENS,), jnp.int32),
        ],
        scratch_shapes=[pltpu.VMEM((HIDDEN, EXPERTS), jnp.float32)],
        compiler_params=pltpu.CompilerParams(
            dimension_semantics=("arbitrary",),
        ),
    )(x, W)
    return (logits, idx)


# speedup vs baseline: 1.1309x; 1.1309x over previous
import jax
import jax.numpy as jnp
from jax.experimental import pallas as pl
from jax.experimental.pallas import tpu as pltpu

TOKENS = 8192
HIDDEN = 2048
EXPERTS = 16
BT = 1024


def _body(x_ref, w_ref, logits_ref, idx_ref, wt_ref):
    @pl.when(pl.program_id(0) == 0)
    def _():
        wt_ref[...] = w_ref[...].T

    xb = x_ref[...]
    l = jnp.dot(xb, wt_ref[...], preferred_element_type=jnp.float32)
    logits_ref[...] = l
    lt = jax.lax.dot_general(
        w_ref[...], xb, (((1,), (1,)), ((), ())), preferred_element_type=jnp.float32
    )
    m = jnp.max(lt, axis=0, keepdims=True)
    e_iota = jax.lax.broadcasted_iota(jnp.int32, (EXPERTS, BT), 0)
    idx_ref[...] = jnp.min(jnp.where(lt == m, e_iota, EXPERTS), axis=0)


def kernel(x, W):
    logits, idx = pl.pallas_call(
        _body,
        grid=(TOKENS // BT,),
        in_specs=[
            pl.BlockSpec((BT, HIDDEN), lambda i: (i, 0)),
            pl.BlockSpec((EXPERTS, HIDDEN), lambda i: (0, 0)),
        ],
        out_specs=[
            pl.BlockSpec((BT, EXPERTS), lambda i: (i, 0)),
            pl.BlockSpec((BT,), lambda i: (i,)),
        ],
        out_shape=[
            jax.ShapeDtypeStruct((TOKENS, EXPERTS), jnp.float32),
            jax.ShapeDtypeStruct((TOKENS,), jnp.int32),
        ],
        scratch_shapes=[pltpu.VMEM((HIDDEN, EXPERTS), jnp.float32)],
        compiler_params=pltpu.CompilerParams(
            dimension_semantics=("arbitrary",),
        ),
    )(x, W)
    return (logits, idx)
